# Initial kernel scaffold; baseline (speedup 1.0000x reference)
import jax
import jax.numpy as jnp
from jax.experimental import pallas as pl


def kernel(x, edge_index, edge_attr, batch, W_enc, b_enc, W_self, W_msg, b_msg, W_dec, b_dec):
    raise NotImplementedError("write your pallas kernel here")


# speedup vs baseline: 5.2923x; 5.2923x over previous
import functools

import jax
import jax.numpy as jnp
from jax import lax
from jax.experimental import pallas as pl
from jax.experimental.pallas import tpu as pltpu
from jax.experimental.pallas import tpu_sc as plsc

N = 50000
E = 800000
IN_C = 7
DIM = 64
OUT_C = 4
HALF = 32
NTEC = 16

CH = 128
CPB = 8
E2 = 802816
PADE = E2 - E
IDX_ROWS = E2 // CH
ROWS_PT_IDX = IDX_ROWS // NTEC
NBLK = ROWS_PT_IDX // CPB
ROWS_PT = 3128
N_SH = ROWS_PT * NTEC
DEG_PT = 3200
DEG_N = DEG_PT * NTEC
DW = 8

_f32 = jnp.float32


_NBLK_TC = 50
_BN = N // _NBLK_TC


def _enc_body(x_ref, we_ref, be_ref, a_ref, y_ref, p_ref):
    y = jnp.dot(x_ref[...], we_ref[...], preferred_element_type=_f32) + be_ref[...]
    z = jnp.dot(y, a_ref[...], preferred_element_type=_f32)
    y_ref[...] = y
    p_ref[...] = jnp.stack([z[:, :HALF], z[:, HALF:]], axis=0)


def _enc(x, W_enc, b_enc, A):
    return pl.pallas_call(
        _enc_body,
        grid=(_NBLK_TC,),
        in_specs=[
            pl.BlockSpec((_BN, IN_C), lambda i: (i, 0)),
            pl.BlockSpec((IN_C, DIM), lambda i: (0, 0)),
            pl.BlockSpec((1, DIM), lambda i: (0, 0)),
            pl.BlockSpec((DIM, DIM), lambda i: (0, 0)),
        ],
        out_specs=[
            pl.BlockSpec((_BN, DIM), lambda i: (i, 0)),
            pl.BlockSpec((2, _BN, HALF), lambda i: (0, i, 0)),
        ],
        out_shape=[
            jax.ShapeDtypeStruct((N, DIM), _f32),
            jax.ShapeDtypeStruct((2, N, HALF), _f32),
        ],
    )(x, W_enc, b_enc, A)


_EBLK = 2048
_NEBLK = E2 // _EBLK


def _hprep_body(ea_ref, b_ref, c_ref, bm_ref, h0_ref, hb_ref):
    ea = ea_ref[...]
    h0 = jnp.dot(ea, b_ref[...], preferred_element_type=_f32) + bm_ref[...]
    hb = h0 - jnp.dot(ea[:, 2:5], c_ref[...], preferred_element_type=_f32)
    h0_ref[...] = jnp.stack([h0[:, :HALF], h0[:, HALF:]], axis=0)
    hb_ref[...] = jnp.stack([hb[:, :HALF], hb[:, HALF:]], axis=0)


def _hprep(ea_pad, B, C3, b_msg):
    return pl.pallas_call(
        _hprep_body,
        grid=(_NEBLK,),
        in_specs=[
            pl.BlockSpec((_EBLK, 8), lambda i: (i, 0)),
            pl.BlockSpec((8, DIM), lambda i: (0, 0)),
            pl.BlockSpec((3, DIM), lambda i: (0, 0)),
            pl.BlockSpec((1, DIM), lambda i: (0, 0)),
        ],
        out_specs=[
            pl.BlockSpec((2, _EBLK, HALF), lambda i: (0, i, 0)),
            pl.BlockSpec((2, _EBLK, HALF), lambda i: (0, i, 0)),
        ],
        out_shape=[
            jax.ShapeDtypeStruct((2, E2, HALF), _f32),
            jax.ShapeDtypeStruct((2, E2, HALF), _f32),
        ],
    )(ea_pad, B, C3, b_msg)


def _dense_body(y_ref, agg_ref, deg_ref, ws_ref, a_ref, wd_ref, c4_ref, bd_ref,
                yn_ref, out_ref, p_ref, q_ref):
    inv = 1.0 / jnp.maximum(deg_ref[...][:, 0:1], 1.0)
    agg = jnp.concatenate([agg_ref[0], agg_ref[1]], axis=1) * inv
    y = y_ref[...]
    y_new = jnp.dot(y, ws_ref[...], preferred_element_type=_f32) + agg + y
    out = jnp.dot(y_new, wd_ref[...], preferred_element_type=_f32) + bd_ref[...]
    u = jnp.dot(out, c4_ref[...], preferred_element_type=_f32)
    p = jnp.dot(y_new, a_ref[...], preferred_element_type=_f32) + u
    yn_ref[...] = y_new
    out_ref[...] = out
    p_ref[...] = jnp.stack([p[:, :HALF], p[:, HALF:]], axis=0)
    q_ref[...] = jnp.stack([u[:, :HALF], u[:, HALF:]], axis=0)


def _dense(y, agg_sh_raw, deg_raw, W_self, A, W_dec, C4, b_dec):
    return pl.pallas_call(
        _dense_body,
        grid=(_NBLK_TC,),
        in_specs=[
            pl.BlockSpec((_BN, DIM), lambda i: (i, 0)),
            pl.BlockSpec((2, _BN, HALF), lambda i: (0, i, 0)),
            pl.BlockSpec((_BN, DW), lambda i: (i, 0)),
            pl.BlockSpec((DIM, DIM), lambda i: (0, 0)),
            pl.BlockSpec((DIM, DIM), lambda i: (0, 0)),
            pl.BlockSpec((DIM, OUT_C), lambda i: (0, 0)),
            pl.BlockSpec((OUT_C, DIM), lambda i: (0, 0)),
            pl.BlockSpec((1, OUT_C), lambda i: (0, 0)),
        ],
        out_specs=[
            pl.BlockSpec((_BN, DIM), lambda i: (i, 0)),
            pl.BlockSpec((_BN, OUT_C), lambda i: (i, 0)),
            pl.BlockSpec((2, _BN, HALF), lambda i: (0, i, 0)),
            pl.BlockSpec((2, _BN, HALF), lambda i: (0, i, 0)),
        ],
        out_shape=[
            jax.ShapeDtypeStruct((N, DIM), _f32),
            jax.ShapeDtypeStruct((N, OUT_C), _f32),
            jax.ShapeDtypeStruct((2, N, HALF), _f32),
            jax.ShapeDtypeStruct((2, N, HALF), _f32),
        ],
    )(y, agg_sh_raw, deg_raw, W_self, A, W_dec, C4, b_dec)


_MESH = plsc.VectorSubcoreMesh(core_axis_name="c", subcore_axis_name="s")
_SC_PARAMS = pltpu.CompilerParams(use_tc_tiling_on_sc=False)


def _sc_deg_body(draw_h, ones_h, z_h, deg_out, dr, ones_v, deg_sh, sem):
    c = lax.axis_index("c")
    s = lax.axis_index("s")
    pltpu.sync_copy(ones_h, ones_v)
    pltpu.sync_copy(z_h, deg_sh.at[pl.ds(s * DEG_PT, DEG_PT)])
    plsc.subcore_barrier()

    def blk(b, carry):
        row0 = s * ROWS_PT_IDX + b * CPB
        pltpu.sync_copy(draw_h.at[pl.ds(row0, CPB)], dr)
        for j in range(CPB):
            pltpu.sync_copy(ones_v, deg_sh.at[dr.at[j]], add=True)
        return carry

    lax.fori_loop(0, NBLK, blk, 0)
    plsc.subcore_barrier()
    pltpu.sync_copy(deg_sh.at[pl.ds(s * DEG_PT, DEG_PT)],
                    deg_out.at[pl.ds(c * DEG_N + s * DEG_PT, DEG_PT)])


def _sc_deg(draw, ones8, zdeg):
    return pl.kernel(
        _sc_deg_body,
        out_type=jax.ShapeDtypeStruct((2 * DEG_N, DW), _f32),
        mesh=_MESH,
        compiler_params=_SC_PARAMS,
        scratch_types=[
            pltpu.VMEM((CPB, CH), jnp.int32),
            pltpu.VMEM((CH, DW), _f32),
            pltpu.VMEM_SHARED((DEG_N, DW), _f32),
            pltpu.SemaphoreType.DMA,
        ],
    )(draw, ones8, zdeg)


def _sc_iter_body(use_q, sadj_h, dadj_h, draw_h, p_h, q_h, h_h, z2_h, agg_out,
                  sb, db, dr, gp, gq, hb, mb, agg_sh, semp, semq, semh):
    c = lax.axis_index("c")
    s = lax.axis_index("s")
    pltpu.sync_copy(z2_h, agg_sh.at[pl.ds(s * ROWS_PT, ROWS_PT)])
    plsc.subcore_barrier()

    def blk(b, carry):
        row0 = s * ROWS_PT_IDX + b * CPB
        pltpu.sync_copy(sadj_h.at[pl.ds(c * IDX_ROWS + row0, CPB)], sb)
        if use_q:
            pltpu.sync_copy(dadj_h.at[pl.ds(c * IDX_ROWS + row0, CPB)], db)
        pltpu.sync_copy(draw_h.at[pl.ds(row0, CPB)], dr)
        for j in range(CPB):
            cp = pltpu.async_copy(p_h.at[sb.at[j]], gp, semp)
            if use_q:
                cq = pltpu.async_copy(q_h.at[db.at[j]], gq, semq)
            hh = pltpu.async_copy(h_h.at[pl.ds(c * E2 + (row0 + j) * CH, CH)], hb, semh)
            cp.wait()
            if use_q:
                cq.wait()
            hh.wait()

            def vrow(i, cc):
                for half in range(2):
                    sl = pl.ds(half * 16, 16)
                    v = gp[i, sl] + hb[i, sl]
                    if use_q:
                        v = v - gq[i, sl]
                    mb[i, sl] = jnp.maximum(v, 0.0)
                return cc

            lax.fori_loop(0, CH, vrow, 0)
            pltpu.sync_copy(mb, agg_sh.at[dr.at[j]], add=True)
        return carry

    lax.fori_loop(0, NBLK, blk, 0)
    plsc.subcore_barrier()
    pltpu.sync_copy(agg_sh.at[pl.ds(s * ROWS_PT, ROWS_PT)],
                    agg_out.at[pl.ds(c * N_SH + s * ROWS_PT, ROWS_PT)])


def _sc_iter(use_q, sadj, dadj, draw, p2, q2, h2, z2):
    return pl.kernel(
        functools.partial(_sc_iter_body, use_q),
        out_type=jax.ShapeDtypeStruct((2 * N_SH, HALF), _f32),
        mesh=_MESH,
        compiler_params=_SC_PARAMS,
        scratch_types=[
            pltpu.VMEM((CPB, CH), jnp.int32),
            pltpu.VMEM((CPB, CH), jnp.int32),
            pltpu.VMEM((CPB, CH), jnp.int32),
            pltpu.VMEM((CH, HALF), _f32),
            pltpu.VMEM((CH, HALF), _f32),
            pltpu.VMEM((CH, HALF), _f32),
            pltpu.VMEM((CH, HALF), _f32),
            pltpu.VMEM_SHARED((N_SH, HALF), _f32),
            pltpu.SemaphoreType.DMA,
            pltpu.SemaphoreType.DMA,
            pltpu.SemaphoreType.DMA,
        ],
    )(sadj, dadj, draw, p2, q2, h2, z2)


def kernel(x, edge_index, edge_attr, batch, W_enc, b_enc, W_self, W_msg, b_msg,
           W_dec, b_dec):
    A = W_msg[:DIM]
    B = W_msg[DIM:]
    C3 = W_msg[DIM + 2:DIM + 5]
    C4 = jnp.concatenate([C3, jnp.zeros((1, DIM), _f32)], axis=0)

    srcf = jnp.pad(edge_index[0], (0, PADE))
    dstg = jnp.pad(edge_index[1], (0, PADE))
    dsts = jnp.pad(edge_index[1], (0, PADE), constant_values=N)
    src2 = srcf.reshape(IDX_ROWS, CH)
    dst2 = dstg.reshape(IDX_ROWS, CH)
    sadj = jnp.concatenate([src2, src2 + N], axis=0)
    dadj = jnp.concatenate([dst2, dst2 + N], axis=0)
    draw = dsts.reshape(IDX_ROWS, CH)

    ones8 = jnp.zeros((CH, DW), _f32).at[:, 0].set(1.0)
    zdeg = jnp.zeros((DEG_PT, DW), _f32)
    z2 = jnp.zeros((ROWS_PT, HALF), _f32)
    ea_pad = jnp.pad(edge_attr, ((0, PADE), (0, 0)))

    y, P = _enc(x, W_enc, b_enc.reshape(1, DIM), A)
    H0, Hb = _hprep(ea_pad, B, C3, b_msg.reshape(1, DIM))
    H0f = H0.reshape(2 * E2, HALF)
    Hbf = Hb.reshape(2 * E2, HALF)

    deg_raw = _sc_deg(draw, ones8, zdeg)

    Pf = P.reshape(2 * N, HALF)
    Qf = Pf
    Hf = H0f
    out = None
    for it in range(3):
        agg = _sc_iter(it > 0, sadj, dadj, draw, Pf, Qf, Hf, z2)
        y, out, Pn, Qn = _dense(y, agg.reshape(2, N_SH, HALF), deg_raw,
                                W_self, A, W_dec, C4, b_dec.reshape(1, OUT_C))
        Pf = Pn.reshape(2 * N, HALF)
        Qf = Qn.reshape(2 * N, HALF)
        Hf = Hbf
    return out


# speedup vs baseline: 5.3144x; 1.0042x over previous
import functools

import jax
import jax.numpy as jnp
from jax import lax
from jax.experimental import pallas as pl
from jax.experimental.pallas import tpu as pltpu
from jax.experimental.pallas import tpu_sc as plsc

N = 50000
E = 800000
IN_C = 7
DIM = 64
OUT_C = 4
HALF = 32
NTEC = 16

CH = 128
CPB = 8
E2 = 802816
PADE = E2 - E
IDX_ROWS = E2 // CH
ROWS_PT_IDX = IDX_ROWS // NTEC
NBLK = ROWS_PT_IDX // CPB
ROWS_PT = 3128
N_SH = ROWS_PT * NTEC
DEG_PT = 3200
DEG_N = DEG_PT * NTEC
DW = 8

_f32 = jnp.float32


_NBLK_TC = 50
_BN = N // _NBLK_TC


def _enc_body(x_ref, we_ref, be_ref, a_ref, y_ref, p_ref):
    y = jnp.dot(x_ref[...], we_ref[...], preferred_element_type=_f32) + be_ref[...]
    z = jnp.dot(y, a_ref[...], preferred_element_type=_f32)
    y_ref[...] = y
    p_ref[...] = jnp.stack([z[:, :HALF], z[:, HALF:]], axis=0)


def _enc(x, W_enc, b_enc, A):
    return pl.pallas_call(
        _enc_body,
        grid=(_NBLK_TC,),
        in_specs=[
            pl.BlockSpec((_BN, IN_C), lambda i: (i, 0)),
            pl.BlockSpec((IN_C, DIM), lambda i: (0, 0)),
            pl.BlockSpec((1, DIM), lambda i: (0, 0)),
            pl.BlockSpec((DIM, DIM), lambda i: (0, 0)),
        ],
        out_specs=[
            pl.BlockSpec((_BN, DIM), lambda i: (i, 0)),
            pl.BlockSpec((2, _BN, HALF), lambda i: (0, i, 0)),
        ],
        out_shape=[
            jax.ShapeDtypeStruct((N, DIM), _f32),
            jax.ShapeDtypeStruct((2, N, HALF), _f32),
        ],
    )(x, W_enc, b_enc, A)


_EBLK = 2000
_NEBLK = E // _EBLK


def _hprep_body(ea_ref, b_ref, c_ref, bm_ref, h0_ref, hb_ref):
    ea = ea_ref[...]
    h0 = jnp.dot(ea, b_ref[...], preferred_element_type=_f32) + bm_ref[...]
    hb = h0 - jnp.dot(ea[:, 2:5], c_ref[...], preferred_element_type=_f32)
    h0_ref[...] = jnp.stack([h0[:, :HALF], h0[:, HALF:]], axis=0)
    hb_ref[...] = jnp.stack([hb[:, :HALF], hb[:, HALF:]], axis=0)


def _hprep(edge_attr, B, C3, b_msg):
    return pl.pallas_call(
        _hprep_body,
        grid=(_NEBLK,),
        in_specs=[
            pl.BlockSpec((_EBLK, 8), lambda i: (i, 0)),
            pl.BlockSpec((8, DIM), lambda i: (0, 0)),
            pl.BlockSpec((3, DIM), lambda i: (0, 0)),
            pl.BlockSpec((1, DIM), lambda i: (0, 0)),
        ],
        out_specs=[
            pl.BlockSpec((2, _EBLK, HALF), lambda i: (0, i, 0)),
            pl.BlockSpec((2, _EBLK, HALF), lambda i: (0, i, 0)),
        ],
        out_shape=[
            jax.ShapeDtypeStruct((2, E2, HALF), _f32),
            jax.ShapeDtypeStruct((2, E2, HALF), _f32),
        ],
    )(edge_attr, B, C3, b_msg)


def _dense_body(y_ref, agg_ref, deg_ref, ws_ref, a_ref, wd_ref, c4_ref, bd_ref,
                yn_ref, out_ref, p_ref, q_ref):
    inv = 1.0 / jnp.maximum(deg_ref[...][:, 0:1], 1.0)
    agg = jnp.concatenate([agg_ref[0], agg_ref[1]], axis=1) * inv
    y = y_ref[...]
    y_new = jnp.dot(y, ws_ref[...], preferred_element_type=_f32) + agg + y
    out = jnp.dot(y_new, wd_ref[...], preferred_element_type=_f32) + bd_ref[...]
    u = jnp.dot(out, c4_ref[...], preferred_element_type=_f32)
    p = jnp.dot(y_new, a_ref[...], preferred_element_type=_f32) + u
    yn_ref[...] = y_new
    out_ref[...] = out
    p_ref[...] = jnp.stack([p[:, :HALF], p[:, HALF:]], axis=0)
    q_ref[...] = jnp.stack([u[:, :HALF], u[:, HALF:]], axis=0)


def _dense(y, agg2, deg_raw, W_self, A, W_dec, C4, b_dec):
    return pl.pallas_call(
        _dense_body,
        grid=(_NBLK_TC,),
        in_specs=[
            pl.BlockSpec((_BN, DIM), lambda i: (i, 0)),
            pl.BlockSpec((2, _BN, HALF), lambda i: (0, i, 0)),
            pl.BlockSpec((_BN, DW), lambda i: (i, 0)),
            pl.BlockSpec((DIM, DIM), lambda i: (0, 0)),
            pl.BlockSpec((DIM, DIM), lambda i: (0, 0)),
            pl.BlockSpec((DIM, OUT_C), lambda i: (0, 0)),
            pl.BlockSpec((OUT_C, DIM), lambda i: (0, 0)),
            pl.BlockSpec((1, OUT_C), lambda i: (0, 0)),
        ],
        out_specs=[
            pl.BlockSpec((_BN, DIM), lambda i: (i, 0)),
            pl.BlockSpec((_BN, OUT_C), lambda i: (i, 0)),
            pl.BlockSpec((2, _BN, HALF), lambda i: (0, i, 0)),
            pl.BlockSpec((2, _BN, HALF), lambda i: (0, i, 0)),
        ],
        out_shape=[
            jax.ShapeDtypeStruct((N, DIM), _f32),
            jax.ShapeDtypeStruct((N, OUT_C), _f32),
            jax.ShapeDtypeStruct((2, N, HALF), _f32),
            jax.ShapeDtypeStruct((2, N, HALF), _f32),
        ],
    )(y, agg2, deg_raw, W_self, A, W_dec, C4, b_dec)


_MESH = plsc.VectorSubcoreMesh(core_axis_name="c", subcore_axis_name="s")
_SC_PARAMS = pltpu.CompilerParams(use_tc_tiling_on_sc=False)


def _sc_deg_body(draw_h, ones_h, z_h, deg_out, dr, ones_v, deg_sh, sem):
    c = lax.axis_index("c")
    s = lax.axis_index("s")
    pltpu.sync_copy(ones_h, ones_v)
    pltpu.sync_copy(z_h, deg_sh.at[pl.ds(s * DEG_PT, DEG_PT)])
    plsc.subcore_barrier()

    def blk(b, carry):
        row0 = s * ROWS_PT_IDX + b * CPB
        pltpu.sync_copy(draw_h.at[pl.ds(row0, CPB)], dr)
        for j in range(CPB):
            pltpu.sync_copy(ones_v, deg_sh.at[dr.at[j]], add=True)
        return carry

    lax.fori_loop(0, NBLK, blk, 0)
    plsc.subcore_barrier()
    pltpu.sync_copy(deg_sh.at[pl.ds(s * DEG_PT, DEG_PT)],
                    deg_out.at[pl.ds(c * DEG_N + s * DEG_PT, DEG_PT)])


def _sc_deg(draw, ones8, zdeg):
    return pl.kernel(
        _sc_deg_body,
        out_type=jax.ShapeDtypeStruct((2 * DEG_N, DW), _f32),
        mesh=_MESH,
        compiler_params=_SC_PARAMS,
        scratch_types=[
            pltpu.VMEM((CPB, CH), jnp.int32),
            pltpu.VMEM((CH, DW), _f32),
            pltpu.VMEM_SHARED((DEG_N, DW), _f32),
            pltpu.SemaphoreType.DMA,
        ],
    )(draw, ones8, zdeg)


def _sc_iter_body(use_q, sgat_h, dgat_h, dsca_h, p_h, q_h, h_h, z2_h, agg_out,
                  sb, db, dr, gp, gq, hb, mb, agg_sh, semp, semq, semh):
    c = lax.axis_index("c")
    s = lax.axis_index("s")
    pltpu.sync_copy(z2_h, agg_sh.at[pl.ds(s * ROWS_PT, ROWS_PT)])
    plsc.subcore_barrier()

    def blk(b, carry):
        row0 = s * ROWS_PT_IDX + b * CPB
        pltpu.sync_copy(sgat_h.at[pl.ds(row0, CPB)], sb)
        if use_q:
            pltpu.sync_copy(dgat_h.at[pl.ds(row0, CPB)], db)
        pltpu.sync_copy(dsca_h.at[pl.ds(row0, CPB)], dr)
        for j in range(CPB):
            cp = pltpu.async_copy(p_h.at[c].at[sb.at[j]], gp, semp)
            if use_q:
                cq = pltpu.async_copy(q_h.at[c].at[db.at[j]], gq, semq)
            hh = pltpu.async_copy(h_h.at[c].at[pl.ds((row0 + j) * CH, CH)], hb, semh)
            cp.wait()
            if use_q:
                cq.wait()
            hh.wait()

            def vrow(i, cc):
                for half in range(2):
                    sl = pl.ds(half * 16, 16)
                    v = gp[i, sl] + hb[i, sl]
                    if use_q:
                        v = v - gq[i, sl]
                    mb[i, sl] = jnp.maximum(v, 0.0)
                return cc

            lax.fori_loop(0, CH, vrow, 0)
            pltpu.sync_copy(mb, agg_sh.at[dr.at[j]], add=True)
        return carry

    lax.fori_loop(0, NBLK, blk, 0)
    plsc.subcore_barrier()
    pltpu.sync_copy(agg_sh.at[pl.ds(s * ROWS_PT, ROWS_PT)],
                    agg_out.at[c].at[pl.ds(s * ROWS_PT, ROWS_PT)])


def _sc_iter(use_q, sgat, dgat, dsca, p2, q2, h2, z2):
    return pl.kernel(
        functools.partial(_sc_iter_body, use_q),
        out_type=jax.ShapeDtypeStruct((2, N_SH, HALF), _f32),
        mesh=_MESH,
        compiler_params=_SC_PARAMS,
        scratch_types=[
            pltpu.VMEM((CPB, CH), jnp.int32),
            pltpu.VMEM((CPB, CH), jnp.int32),
            pltpu.VMEM((CPB, CH), jnp.int32),
            pltpu.VMEM((CH, HALF), _f32),
            pltpu.VMEM((CH, HALF), _f32),
            pltpu.VMEM((CH, HALF), _f32),
            pltpu.VMEM((CH, HALF), _f32),
            pltpu.VMEM_SHARED((N_SH, HALF), _f32),
            pltpu.SemaphoreType.DMA,
            pltpu.SemaphoreType.DMA,
            pltpu.SemaphoreType.DMA,
        ],
    )(sgat, dgat, dsca, p2, q2, h2, z2)


def kernel(x, edge_index, edge_attr, batch, W_enc, b_enc, W_self, W_msg, b_msg,
           W_dec, b_dec):
    A = W_msg[:DIM]
    B = W_msg[DIM:]
    C3 = W_msg[DIM + 2:DIM + 5]
    C4 = jnp.concatenate([C3, jnp.zeros((1, DIM), _f32)], axis=0)

    sgat = jnp.pad(edge_index[0], (0, PADE)).reshape(IDX_ROWS, CH)
    dgat = jnp.pad(edge_index[1], (0, PADE)).reshape(IDX_ROWS, CH)
    dsca = jnp.pad(edge_index[1], (0, PADE),
                   constant_values=N).reshape(IDX_ROWS, CH)

    ones8 = jnp.zeros((CH, DW), _f32).at[:, 0].set(1.0)
    zdeg = jnp.zeros((DEG_PT, DW), _f32)
    z2 = jnp.zeros((ROWS_PT, HALF), _f32)

    y, P = _enc(x, W_enc, b_enc.reshape(1, DIM), A)
    H0, Hb = _hprep(edge_attr, B, C3, b_msg.reshape(1, DIM))

    deg_raw = _sc_deg(dsca, ones8, zdeg)

    Q = P
    H = H0
    out = None
    for it in range(3):
        agg = _sc_iter(it > 0, sgat, dgat, dsca, P, Q, H, z2)
        y, out, P, Q = _dense(y, agg, deg_raw,
                              W_self, A, W_dec, C4, b_dec.reshape(1, OUT_C))
        H = Hb
    return out


# speedup vs baseline: 6.8703x; 1.2928x over previous
import functools

import jax
import jax.numpy as jnp
from jax import lax
from jax.experimental import pallas as pl
from jax.experimental.pallas import tpu as pltpu
from jax.experimental.pallas import tpu_sc as plsc

N = 50000
E = 800000
IN_C = 7
DIM = 64
OUT_C = 4
HALF = 32
NTEC = 16

CH = 128
CPB = 8
_W = 2
E2 = 802816
PADE = E2 - E
IDX_ROWS = E2 // CH
ROWS_PT_IDX = IDX_ROWS // NTEC
NBLK = ROWS_PT_IDX // CPB
ROWS_PT = 3128
N_SH = ROWS_PT * NTEC
DEG_PT = 3200
DEG_N = DEG_PT * NTEC
DW = 8

_f32 = jnp.float32


_NBLK_TC = 50
_BN = N // _NBLK_TC


def _enc_body(x_ref, we_ref, be_ref, a_ref, y_ref, p_ref):
    y = jnp.dot(x_ref[...], we_ref[...], preferred_element_type=_f32) + be_ref[...]
    z = jnp.dot(y, a_ref[...], preferred_element_type=_f32)
    y_ref[...] = y
    p_ref[...] = jnp.stack([z[:, :HALF], z[:, HALF:]], axis=0)


def _enc(x, W_enc, b_enc, A):
    return pl.pallas_call(
        _enc_body,
        grid=(_NBLK_TC,),
        in_specs=[
            pl.BlockSpec((_BN, IN_C), lambda i: (i, 0)),
            pl.BlockSpec((IN_C, DIM), lambda i: (0, 0)),
            pl.BlockSpec((1, DIM), lambda i: (0, 0)),
            pl.BlockSpec((DIM, DIM), lambda i: (0, 0)),
        ],
        out_specs=[
            pl.BlockSpec((_BN, DIM), lambda i: (i, 0)),
            pl.BlockSpec((2, _BN, HALF), lambda i: (0, i, 0)),
        ],
        out_shape=[
            jax.ShapeDtypeStruct((N, DIM), _f32),
            jax.ShapeDtypeStruct((2, N, HALF), _f32),
        ],
    )(x, W_enc, b_enc, A)


_EBLK = 2048
_NEBLK = E2 // _EBLK


_HBLK = 512


def _hprep_body(ea_ref, bb_ref, bz_ref, bias_ref, h0_ref, hb_ref):
    ea = ea_ref[...]
    h0 = jnp.dot(ea, bb_ref[...], preferred_element_type=_f32) + bias_ref[...]
    hb = jnp.dot(ea, bz_ref[...], preferred_element_type=_f32) + bias_ref[...]
    rid = (pl.program_id(0) * _HBLK
           + lax.broadcasted_iota(jnp.int32, (_HBLK, 256), 0))
    h0 = jnp.where(rid < E // 4, h0, -1e30)
    hb = jnp.where(rid < E // 4, hb, -1e30)
    h0_ref[...] = jnp.stack([h0[:, :128], h0[:, 128:]], axis=0)
    hb_ref[...] = jnp.stack([hb[:, :128], hb[:, 128:]], axis=0)


def _hprep(ea2, BB, BZ, bias2):
    return pl.pallas_call(
        _hprep_body,
        grid=(_NEBLK,),
        in_specs=[
            pl.BlockSpec((_HBLK, 32), lambda i: (i, 0)),
            pl.BlockSpec((32, 256), lambda i: (0, 0)),
            pl.BlockSpec((32, 256), lambda i: (0, 0)),
            pl.BlockSpec((1, 256), lambda i: (0, 0)),
        ],
        out_specs=[
            pl.BlockSpec((2, _HBLK, 128), lambda i: (0, i, 0)),
            pl.BlockSpec((2, _HBLK, 128), lambda i: (0, i, 0)),
        ],
        out_shape=[
            jax.ShapeDtypeStruct((2, E2 // 4, 128), _f32),
            jax.ShapeDtypeStruct((2, E2 // 4, 128), _f32),
        ],
    )(ea2, BB, BZ, bias2)


_IBLK = 64
_NIBLK = IDX_ROWS // _IBLK
_IEDG = _IBLK * CH


def _idxprep_body(ei_ref, sg_ref, dg_ref, ds_ref):
    pid = pl.program_id(0)
    src = ei_ref[0].reshape(_IBLK, CH)
    dst = ei_ref[1].reshape(_IBLK, CH)
    ids = (pid * _IEDG
           + lax.broadcasted_iota(jnp.int32, (_IBLK, CH), 0) * CH
           + lax.broadcasted_iota(jnp.int32, (_IBLK, CH), 1))
    mask = ids < E
    sg_ref[...] = jnp.where(mask, src, 0)
    dg_ref[...] = jnp.where(mask, dst, 0)
    ds_ref[...] = jnp.where(mask, dst, N)


def _idxprep(edge_index):
    return pl.pallas_call(
        _idxprep_body,
        grid=(_NIBLK,),
        in_specs=[pl.BlockSpec((2, _IEDG), lambda i: (0, i))],
        out_specs=[
            pl.BlockSpec((_IBLK, CH), lambda i: (i, 0)),
            pl.BlockSpec((_IBLK, CH), lambda i: (i, 0)),
            pl.BlockSpec((_IBLK, CH), lambda i: (i, 0)),
        ],
        out_shape=[
            jax.ShapeDtypeStruct((IDX_ROWS, CH), jnp.int32),
            jax.ShapeDtypeStruct((IDX_ROWS, CH), jnp.int32),
            jax.ShapeDtypeStruct((IDX_ROWS, CH), jnp.int32),
        ],
    )(edge_index)


def _dense_body(y_ref, agg_ref, deg_ref, ws_ref, a_ref, wd_ref, c4_ref, bd_ref,
                yn_ref, out_ref, p_ref, q_ref):
    inv = 1.0 / jnp.maximum(deg_ref[...][:, 0:1], 1.0)
    agg = jnp.concatenate([agg_ref[0], agg_ref[1]], axis=1) * inv
    y = y_ref[...]
    y_new = jnp.dot(y, ws_ref[...], preferred_element_type=_f32) + agg + y
    out = jnp.dot(y_new, wd_ref[...], preferred_element_type=_f32) + bd_ref[...]
    u = jnp.dot(out, c4_ref[...], preferred_element_type=_f32)
    p = jnp.dot(y_new, a_ref[...], preferred_element_type=_f32) + u
    yn_ref[...] = y_new
    out_ref[...] = out
    p_ref[...] = jnp.stack([p[:, :HALF], p[:, HALF:]], axis=0)
    q_ref[...] = jnp.stack([u[:, :HALF], u[:, HALF:]], axis=0)


def _dense(y, agg2, deg_raw, W_self, A, W_dec, C4, b_dec):
    return pl.pallas_call(
        _dense_body,
        grid=(_NBLK_TC,),
        in_specs=[
            pl.BlockSpec((_BN, DIM), lambda i: (i, 0)),
            pl.BlockSpec((2, _BN, HALF), lambda i: (0, i, 0)),
            pl.BlockSpec((_BN, DW), lambda i: (i, 0)),
            pl.BlockSpec((DIM, DIM), lambda i: (0, 0)),
            pl.BlockSpec((DIM, DIM), lambda i: (0, 0)),
            pl.BlockSpec((DIM, OUT_C), lambda i: (0, 0)),
            pl.BlockSpec((OUT_C, DIM), lambda i: (0, 0)),
            pl.BlockSpec((1, OUT_C), lambda i: (0, 0)),
        ],
        out_specs=[
            pl.BlockSpec((_BN, DIM), lambda i: (i, 0)),
            pl.BlockSpec((_BN, OUT_C), lambda i: (i, 0)),
            pl.BlockSpec((2, _BN, HALF), lambda i: (0, i, 0)),
            pl.BlockSpec((2, _BN, HALF), lambda i: (0, i, 0)),
        ],
        out_shape=[
            jax.ShapeDtypeStruct((N, DIM), _f32),
            jax.ShapeDtypeStruct((N, OUT_C), _f32),
            jax.ShapeDtypeStruct((2, N, HALF), _f32),
            jax.ShapeDtypeStruct((2, N, HALF), _f32),
        ],
    )(y, agg2, deg_raw, W_self, A, W_dec, C4, b_dec)


_MESH = plsc.VectorSubcoreMesh(core_axis_name="c", subcore_axis_name="s")
_SC_PARAMS = pltpu.CompilerParams(use_tc_tiling_on_sc=False)


def _sc_deg_body(draw_h, ones_h, z_h, deg_out, dr, ones_v, deg_sh, sem):
    c = lax.axis_index("c")
    s = lax.axis_index("s")
    pltpu.sync_copy(ones_h, ones_v)
    pltpu.sync_copy(z_h, deg_sh.at[pl.ds(s * DEG_PT, DEG_PT)])
    plsc.subcore_barrier()

    def blk(b, carry):
        row0 = s * ROWS_PT_IDX + b * CPB
        pltpu.sync_copy(draw_h.at[pl.ds(row0, CPB)], dr)
        for j in range(CPB):
            pltpu.sync_copy(ones_v, deg_sh.at[dr.at[j]], add=True)
        return carry

    lax.fori_loop(0, NBLK, blk, 0)
    plsc.subcore_barrier()
    pltpu.sync_copy(deg_sh.at[pl.ds(s * DEG_PT, DEG_PT)],
                    deg_out.at[pl.ds(c * DEG_N + s * DEG_PT, DEG_PT)])


def _sc_deg(draw, ones8, zdeg):
    return pl.kernel(
        _sc_deg_body,
        out_type=jax.ShapeDtypeStruct((2 * DEG_N, DW), _f32),
        mesh=_MESH,
        compiler_params=_SC_PARAMS,
        scratch_types=[
            pltpu.VMEM((CPB, CH), jnp.int32),
            pltpu.VMEM((CH, DW), _f32),
            pltpu.VMEM_SHARED((DEG_N, DW), _f32),
            pltpu.SemaphoreType.DMA,
        ],
    )(draw, ones8, zdeg)


def _sc_iter_body(use_q, sgat_h, dgat_h, p_h, q_h, h_h, z2_h, agg_out, *scr):
    if use_q:
        sb, db, gpb, gqb, hbb, mb, agg_sh, semp, semq, semh, sems = scr
    else:
        sb, db, gpb, hbb, mb, agg_sh, semp, semh, sems = scr
        gqb = semq = None
    c = lax.axis_index("c")
    s = lax.axis_index("s")
    pltpu.sync_copy(z2_h, agg_sh.at[pl.ds(s * ROWS_PT, ROWS_PT)])
    plsc.subcore_barrier()

    def blk(b, carry):
        row0 = s * ROWS_PT_IDX + b * CPB
        pltpu.sync_copy(sgat_h.at[pl.ds(row0, CPB)], sb)
        pltpu.sync_copy(dgat_h.at[pl.ds(row0, CPB)], db)

        def fire(j):
            w = j % _W
            cp = pltpu.async_copy(p_h.at[c].at[sb.at[j]], gpb.at[w], semp)
            cq = (pltpu.async_copy(q_h.at[c].at[db.at[j]], gqb.at[w], semq)
                  if use_q else None)
            return (cp, cq)

        for j in range(CPB):
            hh = pltpu.async_copy(
                h_h.at[c].at[pl.ds((row0 + j) * (CH // 4), CH // 4)], hbb, semh)
            cp, cq = fire(j)
            cp.wait()
            if use_q:
                cq.wait()
            hh.wait()
            gp = gpb.at[j % _W]
            gq = gqb.at[j % _W] if use_q else None
            m = mb.at[j % 2]

            def vrow(r, cc):
                for k in range(4):
                    e = r * 4 + k
                    for half in range(2):
                        v = gp[e, pl.ds(half * 16, 16)] + hbb[r, pl.ds(k * 32 + half * 16, 16)]
                        if use_q:
                            v = v - gq[e, pl.ds(half * 16, 16)]
                        m[e, pl.ds(half * 16, 16)] = jnp.maximum(v, 0.0)
                return cc

            lax.fori_loop(0, CH // 4, vrow, 0)
            pltpu.sync_copy(mb.at[j % 2], agg_sh.at[db.at[j]], add=True)
        return carry

    lax.fori_loop(0, NBLK, blk, 0)
    plsc.subcore_barrier()
    pltpu.sync_copy(agg_sh.at[pl.ds(s * ROWS_PT, ROWS_PT)],
                    agg_out.at[c].at[pl.ds(s * ROWS_PT, ROWS_PT)])


def _sc_iter(use_q, sgat, dgat, p2, q2, h2, z2):
    scratch = [
        pltpu.VMEM((CPB, CH), jnp.int32),
        pltpu.VMEM((CPB, CH), jnp.int32),
        pltpu.VMEM((_W, CH, HALF), _f32),
    ]
    if use_q:
        scratch.append(pltpu.VMEM((_W, CH, HALF), _f32))
    scratch += [
        pltpu.VMEM((CH // 4, 128), _f32),
        pltpu.VMEM((2, CH, HALF), _f32),
        pltpu.VMEM_SHARED((N_SH, HALF), _f32),
        pltpu.SemaphoreType.DMA,
    ]
    if use_q:
        scratch.append(pltpu.SemaphoreType.DMA)
    scratch += [
        pltpu.SemaphoreType.DMA,
        pltpu.SemaphoreType.DMA,
    ]
    return pl.kernel(
        functools.partial(_sc_iter_body, use_q),
        out_type=jax.ShapeDtypeStruct((2, N_SH, HALF), _f32),
        mesh=_MESH,
        compiler_params=_SC_PARAMS,
        scratch_types=scratch,
    )(sgat, dgat, p2, q2, h2, z2)


def kernel(x, edge_index, edge_attr, batch, W_enc, b_enc, W_self, W_msg, b_msg,
           W_dec, b_dec):
    A = W_msg[:DIM]
    B = W_msg[DIM:]
    C3 = W_msg[DIM + 2:DIM + 5]
    C4 = jnp.concatenate([C3, jnp.zeros((1, DIM), _f32)], axis=0)

    rowmask = (jnp.arange(8) >= 2) & (jnp.arange(8) < 5)
    Bz = jnp.where(rowmask[:, None], 0.0, B)
    kb = jnp.arange(32) // 8
    eye4 = (kb[:, None] == jnp.arange(4)[None, :]).astype(_f32)

    def _blockdiag(W):
        Wt = jnp.tile(W, (4, 1))
        left = eye4[:, :, None] * Wt[:, None, :HALF]
        right = eye4[:, :, None] * Wt[:, None, HALF:]
        return jnp.concatenate([left.reshape(32, 128),
                                right.reshape(32, 128)], axis=1)

    BB = _blockdiag(B)
    BZ = _blockdiag(Bz)
    bias2 = jnp.concatenate([jnp.tile(b_msg[:HALF], 4),
                             jnp.tile(b_msg[HALF:], 4)]).reshape(1, 256)
    ea2 = jnp.pad(edge_attr.reshape(E // 4, 32), ((0, E2 // 4 - E // 4), (0, 0)))

    sgat, dgat, dsca = _idxprep(jnp.pad(edge_index, ((0, 0), (0, PADE))))

    ones8 = jnp.zeros((CH, DW), _f32).at[:, 0].set(1.0)
    zdeg = jnp.zeros((DEG_PT, DW), _f32)
    z2 = jnp.zeros((ROWS_PT, HALF), _f32)

    y, P = _enc(x, W_enc, b_enc.reshape(1, DIM), A)
    H0, Hb = _hprep(ea2, BB, BZ, bias2)

    deg_raw = _sc_deg(dsca, ones8, zdeg)

    Q = P
    H = H0
    out = None
    for it in range(3):
        agg = _sc_iter(it > 0, sgat, dgat, P, Q, H, z2)
        y, out, P, Q = _dense(y, agg, deg_raw,
                              W_self, A, W_dec, C4, b_dec.reshape(1, OUT_C))
        H = Hb
    return out


# speedup vs baseline: 7.3866x; 1.0752x over previous
import functools

import jax
import jax.numpy as jnp
from jax import lax
from jax.experimental import pallas as pl
from jax.experimental.pallas import tpu as pltpu
from jax.experimental.pallas import tpu_sc as plsc

N = 50000
E = 800000
IN_C = 7
DIM = 64
OUT_C = 4
HALF = 32
NTEC = 16

CH = 128
CPB = 8
_W = 2
E2 = 802816
PADE = E2 - E
IDX_ROWS = E2 // CH
ROWS_PT_IDX = IDX_ROWS // NTEC
NBLK = ROWS_PT_IDX // CPB
ROWS_PT = 3128
N_SH = ROWS_PT * NTEC
DEG_PT = 3200
DEG_N = DEG_PT * NTEC
DW = 8

_f32 = jnp.float32


_NBLK_TC = 50
_BN = N // _NBLK_TC


def _enc_body(x_ref, we_ref, be_ref, a_ref, y_ref, p_ref):
    y = jnp.dot(x_ref[...], we_ref[...], preferred_element_type=_f32) + be_ref[...]
    z = jnp.dot(y, a_ref[...], preferred_element_type=_f32)
    y_ref[...] = y
    p_ref[...] = jnp.stack([z[:, :HALF], z[:, HALF:]], axis=0)


def _enc(x, W_enc, b_enc, A):
    return pl.pallas_call(
        _enc_body,
        grid=(_NBLK_TC,),
        in_specs=[
            pl.BlockSpec((_BN, IN_C), lambda i: (i, 0)),
            pl.BlockSpec((IN_C, DIM), lambda i: (0, 0)),
            pl.BlockSpec((1, DIM), lambda i: (0, 0)),
            pl.BlockSpec((DIM, DIM), lambda i: (0, 0)),
        ],
        out_specs=[
            pl.BlockSpec((_BN, DIM), lambda i: (i, 0)),
            pl.BlockSpec((2, _BN, HALF), lambda i: (0, i, 0)),
        ],
        out_shape=[
            jax.ShapeDtypeStruct((N, DIM), _f32),
            jax.ShapeDtypeStruct((2, N, HALF), _f32),
        ],
    )(x, W_enc, b_enc, A)


_EBLK = 2048
_NEBLK = E2 // _EBLK


_HBLK = 512


def _hprep_body(ea_ref, bb_ref, bz_ref, bias_ref, h0_ref, hb_ref):
    ea = ea_ref[...]
    h0 = jnp.dot(ea, bb_ref[...], preferred_element_type=_f32) + bias_ref[...]
    hb = jnp.dot(ea, bz_ref[...], preferred_element_type=_f32) + bias_ref[...]
    rid = (pl.program_id(0) * _HBLK
           + lax.broadcasted_iota(jnp.int32, (_HBLK, 256), 0))
    h0 = jnp.where(rid < E // 4, h0, -1e30)
    hb = jnp.where(rid < E // 4, hb, -1e30)
    h0_ref[...] = jnp.stack([h0[:, :128], h0[:, 128:]], axis=0)
    hb_ref[...] = jnp.stack([hb[:, :128], hb[:, 128:]], axis=0)


def _hprep(ea2, BB, BZ, bias2):
    return pl.pallas_call(
        _hprep_body,
        grid=(_NEBLK,),
        in_specs=[
            pl.BlockSpec((_HBLK, 32), lambda i: (i, 0)),
            pl.BlockSpec((32, 256), lambda i: (0, 0)),
            pl.BlockSpec((32, 256), lambda i: (0, 0)),
            pl.BlockSpec((1, 256), lambda i: (0, 0)),
        ],
        out_specs=[
            pl.BlockSpec((2, _HBLK, 128), lambda i: (0, i, 0)),
            pl.BlockSpec((2, _HBLK, 128), lambda i: (0, i, 0)),
        ],
        out_shape=[
            jax.ShapeDtypeStruct((2, E2 // 4, 128), _f32),
            jax.ShapeDtypeStruct((2, E2 // 4, 128), _f32),
        ],
    )(ea2, BB, BZ, bias2)


_IBLK = 64
_NIBLK = IDX_ROWS // _IBLK
_IEDG = _IBLK * CH


def _idxprep_body(ei_ref, sg_ref, dg_ref, ds_ref):
    pid = pl.program_id(0)
    src = ei_ref[0].reshape(_IBLK, CH)
    dst = ei_ref[1].reshape(_IBLK, CH)
    ids = (pid * _IEDG
           + lax.broadcasted_iota(jnp.int32, (_IBLK, CH), 0) * CH
           + lax.broadcasted_iota(jnp.int32, (_IBLK, CH), 1))
    mask = ids < E
    sg_ref[...] = jnp.where(mask, src, 0)
    dg_ref[...] = jnp.where(mask, dst, 0)
    ds_ref[...] = jnp.where(mask, dst, N)


def _idxprep(edge_index):
    return pl.pallas_call(
        _idxprep_body,
        grid=(_NIBLK,),
        in_specs=[pl.BlockSpec((2, _IEDG), lambda i: (0, i))],
        out_specs=[
            pl.BlockSpec((_IBLK, CH), lambda i: (i, 0)),
            pl.BlockSpec((_IBLK, CH), lambda i: (i, 0)),
            pl.BlockSpec((_IBLK, CH), lambda i: (i, 0)),
        ],
        out_shape=[
            jax.ShapeDtypeStruct((IDX_ROWS, CH), jnp.int32),
            jax.ShapeDtypeStruct((IDX_ROWS, CH), jnp.int32),
            jax.ShapeDtypeStruct((IDX_ROWS, CH), jnp.int32),
        ],
    )(edge_index)


def _dense_body(y_ref, agg_ref, deg_ref, ws_ref, a_ref, wd_ref, c4_ref, bd_ref,
                yn_ref, out_ref, p_ref, q_ref):
    inv = 1.0 / jnp.maximum(deg_ref[...][:, 0:1], 1.0)
    agg = jnp.concatenate([agg_ref[0], agg_ref[1]], axis=1) * inv
    y = y_ref[...]
    y_new = jnp.dot(y, ws_ref[...], preferred_element_type=_f32) + agg + y
    out = jnp.dot(y_new, wd_ref[...], preferred_element_type=_f32) + bd_ref[...]
    u = jnp.dot(out, c4_ref[...], preferred_element_type=_f32)
    p = jnp.dot(y_new, a_ref[...], preferred_element_type=_f32) + u
    yn_ref[...] = y_new
    out_ref[...] = out
    p_ref[...] = jnp.stack([p[:, :HALF], p[:, HALF:]], axis=0)
    q_ref[...] = jnp.stack([u[:, :HALF], u[:, HALF:]], axis=0)


def _dense(y, agg2, deg_raw, W_self, A, W_dec, C4, b_dec):
    return pl.pallas_call(
        _dense_body,
        grid=(_NBLK_TC,),
        in_specs=[
            pl.BlockSpec((_BN, DIM), lambda i: (i, 0)),
            pl.BlockSpec((2, _BN, HALF), lambda i: (0, i, 0)),
            pl.BlockSpec((_BN, DW), lambda i: (i, 0)),
            pl.BlockSpec((DIM, DIM), lambda i: (0, 0)),
            pl.BlockSpec((DIM, DIM), lambda i: (0, 0)),
            pl.BlockSpec((DIM, OUT_C), lambda i: (0, 0)),
            pl.BlockSpec((OUT_C, DIM), lambda i: (0, 0)),
            pl.BlockSpec((1, OUT_C), lambda i: (0, 0)),
        ],
        out_specs=[
            pl.BlockSpec((_BN, DIM), lambda i: (i, 0)),
            pl.BlockSpec((_BN, OUT_C), lambda i: (i, 0)),
            pl.BlockSpec((2, _BN, HALF), lambda i: (0, i, 0)),
            pl.BlockSpec((2, _BN, HALF), lambda i: (0, i, 0)),
        ],
        out_shape=[
            jax.ShapeDtypeStruct((N, DIM), _f32),
            jax.ShapeDtypeStruct((N, OUT_C), _f32),
            jax.ShapeDtypeStruct((2, N, HALF), _f32),
            jax.ShapeDtypeStruct((2, N, HALF), _f32),
        ],
    )(y, agg2, deg_raw, W_self, A, W_dec, C4, b_dec)


_MESH = plsc.VectorSubcoreMesh(core_axis_name="c", subcore_axis_name="s")
_SC_PARAMS = pltpu.CompilerParams(use_tc_tiling_on_sc=False)


def _sc_deg_body(draw_h, ones_h, z_h, deg_out, dr, ones_v, deg_sh, sem):
    c = lax.axis_index("c")
    s = lax.axis_index("s")
    pltpu.sync_copy(ones_h, ones_v)
    pltpu.sync_copy(z_h, deg_sh.at[pl.ds(s * DEG_PT, DEG_PT)])
    plsc.subcore_barrier()

    def blk(b, carry):
        row0 = s * ROWS_PT_IDX + b * CPB
        pltpu.sync_copy(draw_h.at[pl.ds(row0, CPB)], dr)
        for j in range(CPB):
            pltpu.sync_copy(ones_v, deg_sh.at[dr.at[j]], add=True)
        return carry

    lax.fori_loop(0, NBLK, blk, 0)
    plsc.subcore_barrier()
    pltpu.sync_copy(deg_sh.at[pl.ds(s * DEG_PT, DEG_PT)],
                    deg_out.at[pl.ds(c * DEG_N + s * DEG_PT, DEG_PT)])


def _sc_deg(draw, ones8, zdeg):
    return pl.kernel(
        _sc_deg_body,
        out_type=jax.ShapeDtypeStruct((2 * DEG_N, DW), _f32),
        mesh=_MESH,
        compiler_params=_SC_PARAMS,
        scratch_types=[
            pltpu.VMEM((CPB, CH), jnp.int32),
            pltpu.VMEM((CH, DW), _f32),
            pltpu.VMEM_SHARED((DEG_N, DW), _f32),
            pltpu.SemaphoreType.DMA,
        ],
    )(draw, ones8, zdeg)


def _sc_iter_body(use_q, sgat_h, dgat_h, p_h, q_h, h_h, z2_h, agg_out, *scr):
    if use_q:
        sb, db, gpb, gqb, hbb, mb, agg_sh, semp, semq, semh, sems = scr
    else:
        sb, db, gpb, hbb, mb, agg_sh, semp, semh, sems = scr
        gqb = semq = None
    c = lax.axis_index("c")
    s = lax.axis_index("s")
    pltpu.sync_copy(z2_h, agg_sh.at[pl.ds(s * ROWS_PT, ROWS_PT)])
    plsc.subcore_barrier()

    def blk(b, carry):
        row0 = s * ROWS_PT_IDX + b * CPB
        pltpu.sync_copy(sgat_h.at[pl.ds(row0, CPB)], sb)
        pltpu.sync_copy(dgat_h.at[pl.ds(row0, CPB)], db)

        def fire(j):
            w = j % _W
            cp = pltpu.async_copy(p_h.at[c].at[sb.at[j]], gpb.at[w], semp)
            cq = (pltpu.async_copy(q_h.at[c].at[db.at[j]], gqb.at[w], semq)
                  if use_q else None)
            return (cp, cq)

        gets = {j: fire(j) for j in range(_W)}
        scats = []
        for j in range(CPB):
            hh = pltpu.async_copy(
                h_h.at[c].at[pl.ds((row0 + j) * (CH // 4), CH // 4)], hbb, semh)
            cp, cq = gets[j]
            cp.wait()
            if use_q:
                cq.wait()
            if j >= 2:
                scats[j - 2].wait()
            hh.wait()
            gp = gpb.at[j % _W]
            gq = gqb.at[j % _W] if use_q else None
            m = mb.at[j % 2]

            def vrow(r, cc):
                for k in range(4):
                    e = r * 4 + k
                    for half in range(2):
                        v = gp[e, pl.ds(half * 16, 16)] + hbb[r, pl.ds(k * 32 + half * 16, 16)]
                        if use_q:
                            v = v - gq[e, pl.ds(half * 16, 16)]
                        m[e, pl.ds(half * 16, 16)] = jnp.maximum(v, 0.0)
                return cc

            lax.fori_loop(0, CH // 4, vrow, 0)
            if j + _W < CPB:
                gets[j + _W] = fire(j + _W)
            scats.append(pltpu.async_copy(mb.at[j % 2], agg_sh.at[db.at[j]], sems,
                                          add=True))
        scats[CPB - 2].wait()
        scats[CPB - 1].wait()
        return carry

    lax.fori_loop(0, NBLK, blk, 0)
    plsc.subcore_barrier()
    pltpu.sync_copy(agg_sh.at[pl.ds(s * ROWS_PT, ROWS_PT)],
                    agg_out.at[c].at[pl.ds(s * ROWS_PT, ROWS_PT)])


def _sc_iter(use_q, sgat, dgat, p2, q2, h2, z2):
    scratch = [
        pltpu.VMEM((CPB, CH), jnp.int32),
        pltpu.VMEM((CPB, CH), jnp.int32),
        pltpu.VMEM((_W, CH, HALF), _f32),
    ]
    if use_q:
        scratch.append(pltpu.VMEM((_W, CH, HALF), _f32))
    scratch += [
        pltpu.VMEM((CH // 4, 128), _f32),
        pltpu.VMEM((2, CH, HALF), _f32),
        pltpu.VMEM_SHARED((N_SH, HALF), _f32),
        pltpu.SemaphoreType.DMA,
    ]
    if use_q:
        scratch.append(pltpu.SemaphoreType.DMA)
    scratch += [
        pltpu.SemaphoreType.DMA,
        pltpu.SemaphoreType.DMA,
    ]
    return pl.kernel(
        functools.partial(_sc_iter_body, use_q),
        out_type=jax.ShapeDtypeStruct((2, N_SH, HALF), _f32),
        mesh=_MESH,
        compiler_params=_SC_PARAMS,
        scratch_types=scratch,
    )(sgat, dgat, p2, q2, h2, z2)


def kernel(x, edge_index, edge_attr, batch, W_enc, b_enc, W_self, W_msg, b_msg,
           W_dec, b_dec):
    A = W_msg[:DIM]
    B = W_msg[DIM:]
    C3 = W_msg[DIM + 2:DIM + 5]
    C4 = jnp.concatenate([C3, jnp.zeros((1, DIM), _f32)], axis=0)

    rowmask = (jnp.arange(8) >= 2) & (jnp.arange(8) < 5)
    Bz = jnp.where(rowmask[:, None], 0.0, B)
    kb = jnp.arange(32) // 8
    eye4 = (kb[:, None] == jnp.arange(4)[None, :]).astype(_f32)

    def _blockdiag(W):
        Wt = jnp.tile(W, (4, 1))
        left = eye4[:, :, None] * Wt[:, None, :HALF]
        right = eye4[:, :, None] * Wt[:, None, HALF:]
        return jnp.concatenate([left.reshape(32, 128),
                                right.reshape(32, 128)], axis=1)

    BB = _blockdiag(B)
    BZ = _blockdiag(Bz)
    bias2 = jnp.concatenate([jnp.tile(b_msg[:HALF], 4),
                             jnp.tile(b_msg[HALF:], 4)]).reshape(1, 256)
    ea2 = jnp.pad(edge_attr.reshape(E // 4, 32), ((0, E2 // 4 - E // 4), (0, 0)))

    sgat, dgat, dsca = _idxprep(jnp.pad(edge_index, ((0, 0), (0, PADE))))

    ones8 = jnp.zeros((CH, DW), _f32).at[:, 0].set(1.0)
    zdeg = jnp.zeros((DEG_PT, DW), _f32)
    z2 = jnp.zeros((ROWS_PT, HALF), _f32)

    y, P = _enc(x, W_enc, b_enc.reshape(1, DIM), A)
    H0, Hb = _hprep(ea2, BB, BZ, bias2)

    deg_raw = _sc_deg(dsca, ones8, zdeg)

    Q = P
    H = H0
    out = None
    for it in range(3):
        agg = _sc_iter(it > 0, sgat, dgat, P, Q, H, z2)
        y, out, P, Q = _dense(y, agg, deg_raw,
                              W_self, A, W_dec, C4, b_dec.reshape(1, OUT_C))
        H = Hb
    return out


# speedup vs baseline: 7.4192x; 1.0044x over previous
import functools

import jax
import jax.numpy as jnp
from jax import lax
from jax.experimental import pallas as pl
from jax.experimental.pallas import tpu as pltpu
from jax.experimental.pallas import tpu_sc as plsc

N = 50000
E = 800000
IN_C = 7
DIM = 64
OUT_C = 4
HALF = 32
NTEC = 16

CH = 128
CPB = 8
_W = 2
E2 = 802816
PADE = E2 - E
IDX_ROWS = E2 // CH
ROWS_PT_IDX = IDX_ROWS // NTEC
NBLK = ROWS_PT_IDX // CPB
ROWS_PT = 3128
N_SH = ROWS_PT * NTEC
DEG_PT = 3200
DEG_N = DEG_PT * NTEC
DW = 8

_f32 = jnp.float32


_NBLK_TC = 50
_BN = N // _NBLK_TC


def _enc_body(x_ref, we_ref, be_ref, a_ref, y_ref, p_ref):
    y = jnp.dot(x_ref[...], we_ref[...], preferred_element_type=_f32) + be_ref[...]
    z = jnp.dot(y, a_ref[...], preferred_element_type=_f32)
    y_ref[...] = y
    p_ref[...] = jnp.stack([z[:, :HALF], z[:, HALF:]], axis=0)


def _enc(x, W_enc, b_enc, A):
    return pl.pallas_call(
        _enc_body,
        grid=(_NBLK_TC,),
        in_specs=[
            pl.BlockSpec((_BN, IN_C), lambda i: (i, 0)),
            pl.BlockSpec((IN_C, DIM), lambda i: (0, 0)),
            pl.BlockSpec((1, DIM), lambda i: (0, 0)),
            pl.BlockSpec((DIM, DIM), lambda i: (0, 0)),
        ],
        out_specs=[
            pl.BlockSpec((_BN, DIM), lambda i: (i, 0)),
            pl.BlockSpec((2, _BN, HALF), lambda i: (0, i, 0)),
        ],
        out_shape=[
            jax.ShapeDtypeStruct((N, DIM), _f32),
            jax.ShapeDtypeStruct((2, N, HALF), _f32),
        ],
    )(x, W_enc, b_enc, A)


_EBLK = 2048
_NEBLK = E2 // _EBLK


_HBLK = 512


def _hprep_body(ea_ref, bb_ref, bias_ref, h_ref):
    ea = ea_ref[...]
    h = jnp.dot(ea, bb_ref[...], preferred_element_type=_f32) + bias_ref[...]
    rid = (pl.program_id(0) * _HBLK
           + lax.broadcasted_iota(jnp.int32, (_HBLK, 256), 0))
    h = jnp.where(rid < E // 4, h, -1e30)
    h_ref[...] = jnp.stack([h[:, :128], h[:, 128:]], axis=0)


def _hprep(ea2, BW, bias2):
    return pl.pallas_call(
        _hprep_body,
        grid=(_NEBLK,),
        in_specs=[
            pl.BlockSpec((_HBLK, 32), lambda i: (i, 0)),
            pl.BlockSpec((32, 256), lambda i: (0, 0)),
            pl.BlockSpec((1, 256), lambda i: (0, 0)),
        ],
        out_specs=pl.BlockSpec((2, _HBLK, 128), lambda i: (0, i, 0)),
        out_shape=jax.ShapeDtypeStruct((2, E2 // 4, 128), _f32),
    )(ea2, BW, bias2)


_IBLK = 64
_NIBLK = IDX_ROWS // _IBLK
_IEDG = _IBLK * CH


def _idxprep_body(ei_ref, sg_ref, dg_ref, ds_ref):
    pid = pl.program_id(0)
    src = ei_ref[0].reshape(_IBLK, CH)
    dst = ei_ref[1].reshape(_IBLK, CH)
    ids = (pid * _IEDG
           + lax.broadcasted_iota(jnp.int32, (_IBLK, CH), 0) * CH
           + lax.broadcasted_iota(jnp.int32, (_IBLK, CH), 1))
    mask = ids < E
    sg_ref[...] = jnp.where(mask, src, 0)
    dg_ref[...] = jnp.where(mask, dst, 0)
    ds_ref[...] = jnp.where(mask, dst, N)


def _idxprep(edge_index):
    return pl.pallas_call(
        _idxprep_body,
        grid=(_NIBLK,),
        in_specs=[pl.BlockSpec((2, _IEDG), lambda i: (0, i))],
        out_specs=[
            pl.BlockSpec((_IBLK, CH), lambda i: (i, 0)),
            pl.BlockSpec((_IBLK, CH), lambda i: (i, 0)),
            pl.BlockSpec((_IBLK, CH), lambda i: (i, 0)),
        ],
        out_shape=[
            jax.ShapeDtypeStruct((IDX_ROWS, CH), jnp.int32),
            jax.ShapeDtypeStruct((IDX_ROWS, CH), jnp.int32),
            jax.ShapeDtypeStruct((IDX_ROWS, CH), jnp.int32),
        ],
    )(edge_index)


def _dense_body(y_ref, agg_ref, deg_ref, ws_ref, a_ref, wd_ref, c4_ref, bd_ref,
                yn_ref, out_ref, p_ref, q_ref):
    inv = 1.0 / jnp.maximum(deg_ref[...][:, 0:1], 1.0)
    agg = jnp.concatenate([agg_ref[0], agg_ref[1]], axis=1) * inv
    y = y_ref[...]
    y_new = jnp.dot(y, ws_ref[...], preferred_element_type=_f32) + agg + y
    out = jnp.dot(y_new, wd_ref[...], preferred_element_type=_f32) + bd_ref[...]
    u = jnp.dot(out, c4_ref[...], preferred_element_type=_f32)
    p = jnp.dot(y_new, a_ref[...], preferred_element_type=_f32) + u
    yn_ref[...] = y_new
    out_ref[...] = out
    p_ref[...] = jnp.stack([p[:, :HALF], p[:, HALF:]], axis=0)
    q_ref[...] = jnp.stack([u[:, :HALF], u[:, HALF:]], axis=0)


def _dense(y, agg2, deg_raw, W_self, A, W_dec, C4, b_dec):
    return pl.pallas_call(
        _dense_body,
        grid=(_NBLK_TC,),
        in_specs=[
            pl.BlockSpec((_BN, DIM), lambda i: (i, 0)),
            pl.BlockSpec((2, _BN, HALF), lambda i: (0, i, 0)),
            pl.BlockSpec((_BN, DW), lambda i: (i, 0)),
            pl.BlockSpec((DIM, DIM), lambda i: (0, 0)),
            pl.BlockSpec((DIM, DIM), lambda i: (0, 0)),
            pl.BlockSpec((DIM, OUT_C), lambda i: (0, 0)),
            pl.BlockSpec((OUT_C, DIM), lambda i: (0, 0)),
            pl.BlockSpec((1, OUT_C), lambda i: (0, 0)),
        ],
        out_specs=[
            pl.BlockSpec((_BN, DIM), lambda i: (i, 0)),
            pl.BlockSpec((_BN, OUT_C), lambda i: (i, 0)),
            pl.BlockSpec((2, _BN, HALF), lambda i: (0, i, 0)),
            pl.BlockSpec((2, _BN, HALF), lambda i: (0, i, 0)),
        ],
        out_shape=[
            jax.ShapeDtypeStruct((N, DIM), _f32),
            jax.ShapeDtypeStruct((N, OUT_C), _f32),
            jax.ShapeDtypeStruct((2, N, HALF), _f32),
            jax.ShapeDtypeStruct((2, N, HALF), _f32),
        ],
    )(y, agg2, deg_raw, W_self, A, W_dec, C4, b_dec)


_MESH = plsc.VectorSubcoreMesh(core_axis_name="c", subcore_axis_name="s")
_SC_PARAMS = pltpu.CompilerParams(use_tc_tiling_on_sc=False)


def _sc_deg_body(draw_h, ones_h, z_h, deg_out, dr, ones_v, deg_sh, sem):
    c = lax.axis_index("c")
    s = lax.axis_index("s")
    pltpu.sync_copy(ones_h, ones_v)
    pltpu.sync_copy(z_h, deg_sh.at[pl.ds(s * DEG_PT, DEG_PT)])
    plsc.subcore_barrier()

    def blk(b, carry):
        row0 = s * ROWS_PT_IDX + b * CPB
        pltpu.sync_copy(draw_h.at[pl.ds(row0, CPB)], dr)
        for j in range(CPB):
            pltpu.sync_copy(ones_v, deg_sh.at[dr.at[j]], add=True)
        return carry

    lax.fori_loop(0, NBLK, blk, 0)
    plsc.subcore_barrier()
    pltpu.sync_copy(deg_sh.at[pl.ds(s * DEG_PT, DEG_PT)],
                    deg_out.at[pl.ds(c * DEG_N + s * DEG_PT, DEG_PT)])


def _sc_deg(draw, ones8, zdeg):
    return pl.kernel(
        _sc_deg_body,
        out_type=jax.ShapeDtypeStruct((2 * DEG_N, DW), _f32),
        mesh=_MESH,
        compiler_params=_SC_PARAMS,
        scratch_types=[
            pltpu.VMEM((CPB, CH), jnp.int32),
            pltpu.VMEM((CH, DW), _f32),
            pltpu.VMEM_SHARED((DEG_N, DW), _f32),
            pltpu.SemaphoreType.DMA,
        ],
    )(draw, ones8, zdeg)


def _sc_iter_body(use_q, sgat_h, dgat_h, p_h, q_h, h_h, z2_h, agg_out, *scr):
    if use_q:
        sb, db, gpb, gqb, hbb, mb, agg_sh, semp, semq, semh, sems = scr
    else:
        sb, db, gpb, hbb, mb, agg_sh, semp, semh, sems = scr
        gqb = semq = None
    c = lax.axis_index("c")
    s = lax.axis_index("s")
    pltpu.sync_copy(z2_h, agg_sh.at[pl.ds(s * ROWS_PT, ROWS_PT)])
    plsc.subcore_barrier()

    def blk(b, carry):
        row0 = s * ROWS_PT_IDX + b * CPB
        pltpu.sync_copy(sgat_h.at[pl.ds(row0, CPB)], sb)
        pltpu.sync_copy(dgat_h.at[pl.ds(row0, CPB)], db)

        def fire(j):
            w = j % _W
            cp = pltpu.async_copy(p_h.at[c].at[sb.at[j]], gpb.at[w], semp)
            cq = (pltpu.async_copy(q_h.at[c].at[db.at[j]], gqb.at[w], semq)
                  if use_q else None)
            return (cp, cq)

        gets = {j: fire(j) for j in range(_W)}
        scats = []
        for j in range(CPB):
            hh = pltpu.async_copy(
                h_h.at[c].at[pl.ds((row0 + j) * (CH // 4), CH // 4)], hbb, semh)
            cp, cq = gets[j]
            cp.wait()
            if use_q:
                cq.wait()
            if j >= 2:
                scats[j - 2].wait()
            hh.wait()
            gp = gpb.at[j % _W]
            gq = gqb.at[j % _W] if use_q else None
            m = mb.at[j % 2]

            def vrow(r, cc):
                for k in range(4):
                    e = r * 4 + k
                    for half in range(2):
                        v = gp[e, pl.ds(half * 16, 16)] + hbb[r, pl.ds(k * 32 + half * 16, 16)]
                        if use_q:
                            v = v - gq[e, pl.ds(half * 16, 16)]
                        m[e, pl.ds(half * 16, 16)] = jnp.maximum(v, 0.0)
                return cc

            lax.fori_loop(0, CH // 4, vrow, 0)
            if j + _W < CPB:
                gets[j + _W] = fire(j + _W)
            scats.append(pltpu.async_copy(mb.at[j % 2], agg_sh.at[db.at[j]], sems,
                                          add=True))
        scats[CPB - 2].wait()
        scats[CPB - 1].wait()
        return carry

    lax.fori_loop(0, NBLK, blk, 0)
    plsc.subcore_barrier()
    pltpu.sync_copy(agg_sh.at[pl.ds(s * ROWS_PT, ROWS_PT)],
                    agg_out.at[c].at[pl.ds(s * ROWS_PT, ROWS_PT)])


def _sc_iter(use_q, sgat, dgat, p2, q2, h2, z2):
    scratch = [
        pltpu.VMEM((CPB, CH), jnp.int32),
        pltpu.VMEM((CPB, CH), jnp.int32),
        pltpu.VMEM((_W, CH, HALF), _f32),
    ]
    if use_q:
        scratch.append(pltpu.VMEM((_W, CH, HALF), _f32))
    scratch += [
        pltpu.VMEM((CH // 4, 128), _f32),
        pltpu.VMEM((2, CH, HALF), _f32),
        pltpu.VMEM_SHARED((N_SH, HALF), _f32),
        pltpu.SemaphoreType.DMA,
    ]
    if use_q:
        scratch.append(pltpu.SemaphoreType.DMA)
    scratch += [
        pltpu.SemaphoreType.DMA,
        pltpu.SemaphoreType.DMA,
    ]
    return pl.kernel(
        functools.partial(_sc_iter_body, use_q),
        out_type=jax.ShapeDtypeStruct((2, N_SH, HALF), _f32),
        mesh=_MESH,
        compiler_params=_SC_PARAMS,
        scratch_types=scratch,
    )(sgat, dgat, p2, q2, h2, z2)


def kernel(x, edge_index, edge_attr, batch, W_enc, b_enc, W_self, W_msg, b_msg,
           W_dec, b_dec):
    A = W_msg[:DIM]
    B = W_msg[DIM:]
    C3 = W_msg[DIM + 2:DIM + 5]
    C4 = jnp.concatenate([C3, jnp.zeros((1, DIM), _f32)], axis=0)

    rowmask = (jnp.arange(8) >= 2) & (jnp.arange(8) < 5)
    Bz = jnp.where(rowmask[:, None], 0.0, B)
    kb = jnp.arange(32) // 8
    eye4 = (kb[:, None] == jnp.arange(4)[None, :]).astype(_f32)

    def _blockdiag(W):
        Wt = jnp.tile(W, (4, 1))
        left = eye4[:, :, None] * Wt[:, None, :HALF]
        right = eye4[:, :, None] * Wt[:, None, HALF:]
        return jnp.concatenate([left.reshape(32, 128),
                                right.reshape(32, 128)], axis=1)

    BB = _blockdiag(B)
    BZ = _blockdiag(Bz)
    bias2 = jnp.concatenate([jnp.tile(b_msg[:HALF], 4),
                             jnp.tile(b_msg[HALF:], 4)]).reshape(1, 256)
    ea2 = jnp.pad(edge_attr.reshape(E // 4, 32), ((0, E2 // 4 - E // 4), (0, 0)))

    sgat, dgat, dsca = _idxprep(jnp.pad(edge_index, ((0, 0), (0, PADE))))

    ones8 = jnp.zeros((CH, DW), _f32).at[:, 0].set(1.0)
    zdeg = jnp.zeros((DEG_PT, DW), _f32)
    z2 = jnp.zeros((ROWS_PT, HALF), _f32)

    y, P = _enc(x, W_enc, b_enc.reshape(1, DIM), A)
    H0 = _hprep(ea2, BB, bias2)
    Hb = _hprep(ea2, BZ, bias2)

    deg_raw = _sc_deg(dsca, ones8, zdeg)

    Q = P
    H = H0
    out = None
    for it in range(3):
        agg = _sc_iter(it > 0, sgat, dgat, P, Q, H, z2)
        y, out, P, Q = _dense(y, agg, deg_raw,
                              W_self, A, W_dec, C4, b_dec.reshape(1, OUT_C))
        H = Hb
    return out
